# baseline (speedup 1.0000x reference)
import functools

import jax
import jax.numpy as jnp
from jax import lax
from jax.experimental import pallas as pl
from jax.experimental.pallas import tpu as pltpu
from jax.experimental.pallas import tpu_sc as plsc

V = 100002
ROWS, COLS = 4096, 2048
N_TOKENS = ROWS * COLS
NC, NS, L = 2, 16, 16
NW = NC * NS
ROWS_W = ROWS // NW
CR, CC = 8, 1024
CHUNK = CR * CC
NCH = ROWS_W // CR * 2
G = 16
RCOLS = 4096
V_PAD = 102400

_mesh = plsc.VectorSubcoreMesh(core_axis_name="c", subcore_axis_name="s")


@functools.partial(
    pl.kernel,
    out_type=jax.ShapeDtypeStruct((NW, V_PAD), jnp.float32),
    mesh=_mesh,
    scratch_types=[
        pltpu.VMEM((CR, CC), jnp.int32),
        pltpu.VMEM((CR, CC), jnp.int32),
        pltpu.VMEM((V_PAD,), jnp.float32),
        pltpu.SemaphoreType.DMA,
        pltpu.SemaphoreType.DMA,
    ],
    compiler_params=pltpu.CompilerParams(
        needs_layout_passes=False, use_tc_tiling_on_sc=True
    ),
)
def _hist_kernel(text_hbm, out_hbm, buf0, buf1, hist, sem0, sem1):
    wid = lax.axis_index("s") * NC + lax.axis_index("c")
    base_row = wid * ROWS_W
    bufs = (buf0, buf1)
    sems = (sem0, sem1)

    zeros = jnp.zeros((L,), jnp.float32)

    def zero_body(i, carry):
        hist[pl.ds(i * L, L)] = zeros
        return carry

    lax.fori_loop(0, V_PAD // L, zero_body, 0, unroll=8)

    def chunk_copy(g, b):
        return pltpu.make_async_copy(
            text_hbm.at[pl.ds(base_row + g * CR, CR), pl.ds(b * CC, CC)],
            bufs[b],
            sems[b],
        )

    chunk_copy(0, 0).start()
    chunk_copy(0, 1).start()

    GROUPS_PER_IT = G * L

    def outer(g, carry):
        for b in range(2):
            chunk_copy(g, b).wait()

            def proc(i, cc, _b=b):
                row = i // (CC // GROUPS_PER_IT)
                colbase = (i % (CC // GROUPS_PER_IT)) * GROUPS_PER_IT
                xs = [
                    bufs[_b][row, pl.ds(colbase + j * L, L)] for j in range(G)
                ]
                crs = [plsc.scan_count(x) for x in xs]
                for x, (cnt, last) in zip(xs, crs):
                    plsc.addupdate_scatter(
                        hist, [x], cnt.astype(jnp.float32), mask=last
                    )
                return cc

            lax.fori_loop(0, CHUNK // GROUPS_PER_IT, proc, 0)

            @pl.when(g + 1 < NCH // 2)
            def _(_g=g, _b=b):
                chunk_copy(_g + 1, _b).start()

        return carry

    lax.fori_loop(0, NCH // 2, outer, 0)

    pltpu.sync_copy(hist, out_hbm.at[wid])


def _reduce_body(parts_ref, unary_ref, out_ref):
    ones = jnp.ones((1, NW), jnp.float32)
    s = jax.lax.dot_general(
        ones,
        parts_ref[...],
        (((1,), (0,)), ((), ())),
        preferred_element_type=jnp.float32,
    )
    out_ref[0] = unary_ref[0] + s


_reduce = pl.pallas_call(
    _reduce_body,
    grid=(V_PAD // RCOLS,),
    in_specs=[
        pl.BlockSpec((NW, RCOLS), lambda i: (0, i)),
        pl.BlockSpec((1, 1, RCOLS), lambda i: (i, 0, 0)),
    ],
    out_specs=pl.BlockSpec((1, 1, RCOLS), lambda i: (i, 0, 0)),
    out_shape=jax.ShapeDtypeStruct((V_PAD // RCOLS, 1, RCOLS), jnp.float32),
)


@jax.jit
def kernel(text, unary_counts):
    parts = _hist_kernel(text)
    unary_pad = jnp.pad(unary_counts, (0, V_PAD - V)).reshape(
        V_PAD // RCOLS, 1, RCOLS
    )
    out = _reduce(parts, unary_pad)
    return out.reshape(-1)[:V]


# speedup vs baseline: 1.2385x; 1.2385x over previous
import functools

import jax
import jax.numpy as jnp
from jax import lax
from jax.experimental import pallas as pl
from jax.experimental.pallas import tpu as pltpu
from jax.experimental.pallas import tpu_sc as plsc

V = 100002
ROWS, COLS = 4096, 2048
N_TOKENS = ROWS * COLS
NC, NS, L = 2, 16, 16
NW = NC * NS
ROWS_W = ROWS // NW
CR, CC = 8, 1024
CHUNK = CR * CC
NCH = ROWS_W // CR * 2
G = 16
RCOLS = 10240
V_PAD = 102400

_mesh = plsc.VectorSubcoreMesh(core_axis_name="c", subcore_axis_name="s")


@functools.partial(
    pl.kernel,
    out_type=jax.ShapeDtypeStruct((NW, V_PAD), jnp.int32),
    mesh=_mesh,
    scratch_types=[
        pltpu.VMEM((CR, CC), jnp.int32),
        pltpu.VMEM((CR, CC), jnp.int32),
        pltpu.VMEM((V_PAD,), jnp.int32),
        pltpu.SemaphoreType.DMA,
        pltpu.SemaphoreType.DMA,
    ],
    compiler_params=pltpu.CompilerParams(
        needs_layout_passes=False, use_tc_tiling_on_sc=True
    ),
)
def _hist_kernel(text_hbm, out_hbm, buf0, buf1, hist, sem0, sem1):
    wid = lax.axis_index("s") * NC + lax.axis_index("c")
    base_row = wid * ROWS_W
    bufs = (buf0, buf1)
    sems = (sem0, sem1)

    zeros = jnp.zeros((L,), jnp.int32)

    def zero_body(i, carry):
        hist[pl.ds(i * L, L)] = zeros
        return carry

    lax.fori_loop(0, V_PAD // L, zero_body, 0, unroll=8)

    def chunk_copy(g, b):
        return pltpu.make_async_copy(
            text_hbm.at[pl.ds(base_row + g * CR, CR), pl.ds(b * CC, CC)],
            bufs[b],
            sems[b],
        )

    chunk_copy(0, 0).start()
    chunk_copy(0, 1).start()

    GROUPS_PER_IT = G * L

    def outer(g, carry):
        for b in range(2):
            chunk_copy(g, b).wait()

            def proc(i, cc, _b=b):
                row = i // (CC // GROUPS_PER_IT)
                colbase = (i % (CC // GROUPS_PER_IT)) * GROUPS_PER_IT
                xs = [
                    bufs[_b][row, pl.ds(colbase + j * L, L)] for j in range(G)
                ]
                crs = [plsc.scan_count(x) for x in xs]
                for x, (cnt, last) in zip(xs, crs):
                    plsc.addupdate_scatter(hist, [x], cnt, mask=last)
                return cc

            lax.fori_loop(0, CHUNK // GROUPS_PER_IT, proc, 0)

            @pl.when(g + 1 < NCH // 2)
            def _(_g=g, _b=b):
                chunk_copy(_g + 1, _b).start()

        return carry

    lax.fori_loop(0, NCH // 2, outer, 0)

    pltpu.sync_copy(hist, out_hbm.at[wid])


def _reduce_body(parts_ref, unary_ref, out_ref):
    s = jnp.sum(parts_ref[...], axis=0, keepdims=True).astype(jnp.float32)
    out_ref[0] = unary_ref[0] + s


_reduce = pl.pallas_call(
    _reduce_body,
    grid=(V_PAD // RCOLS,),
    in_specs=[
        pl.BlockSpec((NW, RCOLS), lambda i: (0, i)),
        pl.BlockSpec((1, 1, RCOLS), lambda i: (i, 0, 0)),
    ],
    out_specs=pl.BlockSpec((1, 1, RCOLS), lambda i: (i, 0, 0)),
    out_shape=jax.ShapeDtypeStruct((V_PAD // RCOLS, 1, RCOLS), jnp.float32),
)


@jax.jit
def kernel(text, unary_counts):
    parts = _hist_kernel(text)
    unary_pad = jnp.pad(unary_counts, (0, V_PAD - V)).reshape(
        V_PAD // RCOLS, 1, RCOLS
    )
    out = _reduce(parts, unary_pad)
    return out.reshape(-1)[:V]


# speedup vs baseline: 1.2786x; 1.0324x over previous
import functools

import jax
import jax.numpy as jnp
from jax import lax
from jax.experimental import pallas as pl
from jax.experimental.pallas import tpu as pltpu
from jax.experimental.pallas import tpu_sc as plsc

V = 100002
ROWS, COLS = 4096, 2048
N_TOKENS = ROWS * COLS
NC, NS, L = 2, 16, 16
NW = NC * NS
ROWS_W = ROWS // NW
CR, CC = 8, 1024
CHUNK = CR * CC
NCH = ROWS_W // CR * 2
G = 16
RCOLS = 20480
V_PAD = 102400

_mesh = plsc.VectorSubcoreMesh(core_axis_name="c", subcore_axis_name="s")


@functools.partial(
    pl.kernel,
    out_type=jax.ShapeDtypeStruct((NW, V_PAD), jnp.int32),
    mesh=_mesh,
    scratch_types=[
        pltpu.VMEM((CR, CC), jnp.int32),
        pltpu.VMEM((CR, CC), jnp.int32),
        pltpu.VMEM((V_PAD,), jnp.int32),
        pltpu.SemaphoreType.DMA,
        pltpu.SemaphoreType.DMA,
    ],
    compiler_params=pltpu.CompilerParams(
        needs_layout_passes=False, use_tc_tiling_on_sc=True
    ),
)
def _hist_kernel(text_hbm, out_hbm, buf0, buf1, hist, sem0, sem1):
    wid = lax.axis_index("s") * NC + lax.axis_index("c")
    base_row = wid * ROWS_W
    bufs = (buf0, buf1)
    sems = (sem0, sem1)

    zeros = jnp.zeros((L,), jnp.int32)

    def zero_body(i, carry):
        hist[pl.ds(i * L, L)] = zeros
        return carry

    lax.fori_loop(0, V_PAD // L, zero_body, 0, unroll=8)

    def chunk_copy(g, b):
        return pltpu.make_async_copy(
            text_hbm.at[pl.ds(base_row + g * CR, CR), pl.ds(b * CC, CC)],
            bufs[b],
            sems[b],
        )

    chunk_copy(0, 0).start()
    chunk_copy(0, 1).start()

    GROUPS_PER_IT = G * L

    def outer(g, carry):
        for b in range(2):
            chunk_copy(g, b).wait()

            def proc(i, cc, _b=b):
                row = i // (CC // GROUPS_PER_IT)
                colbase = (i % (CC // GROUPS_PER_IT)) * GROUPS_PER_IT
                xs = [
                    bufs[_b][row, pl.ds(colbase + j * L, L)] for j in range(G)
                ]
                crs = [plsc.scan_count(x) for x in xs]
                for x, (cnt, last) in zip(xs, crs):
                    plsc.addupdate_scatter(hist, [x], cnt, mask=last)
                return cc

            lax.fori_loop(0, CHUNK // GROUPS_PER_IT, proc, 0)

            @pl.when(g + 1 < NCH // 2)
            def _(_g=g, _b=b):
                chunk_copy(_g + 1, _b).start()

        return carry

    lax.fori_loop(0, NCH // 2, outer, 0)

    pltpu.sync_copy(hist, out_hbm.at[wid])


def _reduce_body(parts_ref, unary_ref, out_ref):
    s = jnp.sum(parts_ref[...], axis=0, keepdims=True).astype(jnp.float32)
    out_ref[0] = unary_ref[0] + s


_reduce = pl.pallas_call(
    _reduce_body,
    grid=(V_PAD // RCOLS,),
    in_specs=[
        pl.BlockSpec((NW, RCOLS), lambda i: (0, i)),
        pl.BlockSpec((1, 1, RCOLS), lambda i: (i, 0, 0)),
    ],
    out_specs=pl.BlockSpec((1, 1, RCOLS), lambda i: (i, 0, 0)),
    out_shape=jax.ShapeDtypeStruct((V_PAD // RCOLS, 1, RCOLS), jnp.float32),
)


@jax.jit
def kernel(text, unary_counts):
    parts = _hist_kernel(text)
    unary_pad = jnp.pad(unary_counts, (0, V_PAD - V)).reshape(
        V_PAD // RCOLS, 1, RCOLS
    )
    out = _reduce(parts, unary_pad)
    return out.reshape(-1)[:V]


# speedup vs baseline: 1.3415x; 1.0492x over previous
import functools

import jax
import jax.numpy as jnp
from jax import lax
from jax.experimental import pallas as pl
from jax.experimental.pallas import tpu as pltpu
from jax.experimental.pallas import tpu_sc as plsc

V = 100002
ROWS, COLS = 4096, 2048
N_TOKENS = ROWS * COLS
NC, NS, L = 2, 16, 16
NW = NC * NS
ROWS_W = ROWS // NW
CR, CC = 8, 1024
CHUNK = CR * CC
NCH = ROWS_W // CR * 2
G = 16
RCOLS = 20480
V_PAD = 102400

_mesh = plsc.VectorSubcoreMesh(core_axis_name="c", subcore_axis_name="s")


@functools.partial(
    pl.kernel,
    out_type=jax.ShapeDtypeStruct((NW, V_PAD), jnp.int32),
    mesh=_mesh,
    scratch_types=[
        pltpu.VMEM((CR, CC), jnp.int32),
        pltpu.VMEM((CR, CC), jnp.int32),
        pltpu.VMEM((V_PAD,), jnp.int32),
        pltpu.SemaphoreType.DMA,
        pltpu.SemaphoreType.DMA,
    ],
    compiler_params=pltpu.CompilerParams(
        needs_layout_passes=False, use_tc_tiling_on_sc=True
    ),
)
def _hist_kernel(text_hbm, out_hbm, buf0, buf1, hist, sem0, sem1):
    wid = lax.axis_index("s") * NC + lax.axis_index("c")
    base_row = wid * ROWS_W
    bufs = (buf0, buf1)
    sems = (sem0, sem1)

    zeros = jnp.zeros((L,), jnp.int32)

    def zero_body(i, carry):
        hist[pl.ds(i * L, L)] = zeros
        return carry

    lax.fori_loop(0, V_PAD // L, zero_body, 0, unroll=8)

    def chunk_copy(g, b):
        return pltpu.make_async_copy(
            text_hbm.at[pl.ds(base_row + g * CR, CR), pl.ds(b * CC, CC)],
            bufs[b],
            sems[b],
        )

    chunk_copy(0, 0).start()
    chunk_copy(0, 1).start()

    GROUPS_PER_IT = G * L

    def outer(g, carry):
        for b in range(2):
            chunk_copy(g, b).wait()

            def proc(i, cc, _b=b):
                row = i // (CC // GROUPS_PER_IT)
                colbase = (i % (CC // GROUPS_PER_IT)) * GROUPS_PER_IT
                xs = [
                    bufs[_b][row, pl.ds(colbase + j * L, L)] for j in range(G)
                ]
                ones = jnp.ones((L,), jnp.int32)
                for x in xs:
                    plsc.addupdate_scatter(hist, [x], ones)
                return cc

            lax.fori_loop(0, CHUNK // GROUPS_PER_IT, proc, 0)

            @pl.when(g + 1 < NCH // 2)
            def _(_g=g, _b=b):
                chunk_copy(_g + 1, _b).start()

        return carry

    lax.fori_loop(0, NCH // 2, outer, 0)

    pltpu.sync_copy(hist, out_hbm.at[wid])


def _reduce_body(parts_ref, unary_ref, out_ref):
    s = jnp.sum(parts_ref[...], axis=0, keepdims=True).astype(jnp.float32)
    out_ref[0] = unary_ref[0] + s


_reduce = pl.pallas_call(
    _reduce_body,
    grid=(V_PAD // RCOLS,),
    in_specs=[
        pl.BlockSpec((NW, RCOLS), lambda i: (0, i)),
        pl.BlockSpec((1, 1, RCOLS), lambda i: (i, 0, 0)),
    ],
    out_specs=pl.BlockSpec((1, 1, RCOLS), lambda i: (i, 0, 0)),
    out_shape=jax.ShapeDtypeStruct((V_PAD // RCOLS, 1, RCOLS), jnp.float32),
)


@jax.jit
def kernel(text, unary_counts):
    parts = _hist_kernel(text)
    unary_pad = jnp.pad(unary_counts, (0, V_PAD - V)).reshape(
        V_PAD // RCOLS, 1, RCOLS
    )
    out = _reduce(parts, unary_pad)
    return out.reshape(-1)[:V]


# speedup vs baseline: 1.4349x; 1.0696x over previous
import functools

import jax
import jax.numpy as jnp
from jax import lax
from jax.experimental import pallas as pl
from jax.experimental.pallas import tpu as pltpu
from jax.experimental.pallas import tpu_sc as plsc

V = 100002
ROWS, COLS = 4096, 2048
N_TOKENS = ROWS * COLS
NC, NS, L = 2, 16, 16
NW = NC * NS
ROWS_W = ROWS // NW
CR, CC = 8, 1024
CHUNK = CR * CC
NCH = ROWS_W // CR * 2
G = 16
RCOLS = 20480
V_PAD = 102400

_mesh = plsc.VectorSubcoreMesh(core_axis_name="c", subcore_axis_name="s")


@functools.partial(
    pl.kernel,
    out_type=jax.ShapeDtypeStruct((NW, V_PAD), jnp.int32),
    mesh=_mesh,
    scratch_types=[
        pltpu.VMEM((CR, CC), jnp.int32),
        pltpu.VMEM((CR, CC), jnp.int32),
        pltpu.VMEM((CR, CC), jnp.int32),
        pltpu.VMEM((V_PAD,), jnp.int32),
        pltpu.SemaphoreType.DMA,
        pltpu.SemaphoreType.DMA,
        pltpu.SemaphoreType.DMA,
    ],
    compiler_params=pltpu.CompilerParams(
        needs_layout_passes=False, use_tc_tiling_on_sc=True
    ),
)
def _hist_kernel(text_hbm, out_hbm, buf0, buf1, buf2, hist, sem0, sem1, sem2):
    wid = lax.axis_index("s") * NC + lax.axis_index("c")
    base_row = wid * ROWS_W
    bufs = (buf0, buf1, buf2)
    sems = (sem0, sem1, sem2)

    zeros = jnp.zeros((L,), jnp.int32)

    def zero_body(i, carry):
        hist[pl.ds(i * L, L)] = zeros
        return carry

    lax.fori_loop(0, V_PAD // L, zero_body, 0, unroll=8)

    def chunk_copy(c, b):
        return pltpu.make_async_copy(
            text_hbm.at[
                pl.ds(base_row + (c // 2) * CR, CR), pl.ds((c % 2) * CC, CC)
            ],
            bufs[b],
            sems[b],
        )

    GROUPS_PER_IT = G * L
    ones = jnp.ones((L,), jnp.int32)

    def process(b):
        def proc(i, cc, _b=b):
            row = i // (CC // GROUPS_PER_IT)
            colbase = (i % (CC // GROUPS_PER_IT)) * GROUPS_PER_IT
            xs = [bufs[_b][row, pl.ds(colbase + j * L, L)] for j in range(G)]
            for x in xs:
                plsc.addupdate_scatter(hist, [x], ones)
            return cc

        lax.fori_loop(0, CHUNK // GROUPS_PER_IT, proc, 0)

    for b in range(3):
        chunk_copy(b, b).start()

    NB = NCH // 3

    def outer(g, carry):
        for b in range(3):
            c = 3 * g + b
            chunk_copy(c, b).wait()
            process(b)

            @pl.when(c + 3 < NCH)
            def _(_c=c, _b=b):
                chunk_copy(_c + 3, _b).start()

        return carry

    lax.fori_loop(0, NB, outer, 0)

    for b in range(NCH - 3 * NB):
        chunk_copy(3 * NB + b, b).wait()
        process(b)

    pltpu.sync_copy(hist, out_hbm.at[wid])


def _reduce_body(parts_ref, unary_ref, out_ref):
    s = jnp.sum(parts_ref[...], axis=0, keepdims=True).astype(jnp.float32)
    out_ref[0] = unary_ref[0] + s


_reduce = pl.pallas_call(
    _reduce_body,
    grid=(V_PAD // RCOLS,),
    in_specs=[
        pl.BlockSpec((NW, RCOLS), lambda i: (0, i)),
        pl.BlockSpec((1, 1, RCOLS), lambda i: (i, 0, 0)),
    ],
    out_specs=pl.BlockSpec((1, 1, RCOLS), lambda i: (i, 0, 0)),
    out_shape=jax.ShapeDtypeStruct((V_PAD // RCOLS, 1, RCOLS), jnp.float32),
)


@jax.jit
def kernel(text, unary_counts):
    parts = _hist_kernel(text)
    unary_pad = jnp.pad(unary_counts, (0, V_PAD - V)).reshape(
        V_PAD // RCOLS, 1, RCOLS
    )
    out = _reduce(parts, unary_pad)
    return out.reshape(-1)[:V]


# speedup vs baseline: 1.4519x; 1.0119x over previous
import functools

import jax
import jax.numpy as jnp
from jax import lax
from jax.experimental import pallas as pl
from jax.experimental.pallas import tpu as pltpu
from jax.experimental.pallas import tpu_sc as plsc

V = 100002
ROWS, COLS = 4096, 2048
N_TOKENS = ROWS * COLS
NC, NS, L = 2, 16, 16
NW = NC * NS
ROWS_W = ROWS // NW
CR, CC = 8, 1024
CHUNK = CR * CC
NCH = ROWS_W // CR * 2
G = 16
RCOLS = 25600
V_PAD = 102400

_mesh = plsc.VectorSubcoreMesh(core_axis_name="c", subcore_axis_name="s")


@functools.partial(
    pl.kernel,
    out_type=jax.ShapeDtypeStruct((NW, V_PAD), jnp.int32),
    mesh=_mesh,
    scratch_types=[
        pltpu.VMEM((CR, CC), jnp.int32),
        pltpu.VMEM((CR, CC), jnp.int32),
        pltpu.VMEM((CR, CC), jnp.int32),
        pltpu.VMEM((V_PAD,), jnp.int32),
        pltpu.SemaphoreType.DMA,
        pltpu.SemaphoreType.DMA,
        pltpu.SemaphoreType.DMA,
    ],
    compiler_params=pltpu.CompilerParams(
        needs_layout_passes=False, use_tc_tiling_on_sc=True
    ),
)
def _hist_kernel(text_hbm, out_hbm, buf0, buf1, buf2, hist, sem0, sem1, sem2):
    wid = lax.axis_index("s") * NC + lax.axis_index("c")
    base_row = wid * ROWS_W
    bufs = (buf0, buf1, buf2)
    sems = (sem0, sem1, sem2)

    zeros = jnp.zeros((L,), jnp.int32)

    def zero_body(i, carry):
        hist[pl.ds(i * L, L)] = zeros
        return carry

    lax.fori_loop(0, V_PAD // L, zero_body, 0, unroll=8)

    def chunk_copy(c, b):
        return pltpu.make_async_copy(
            text_hbm.at[
                pl.ds(base_row + (c // 2) * CR, CR), pl.ds((c % 2) * CC, CC)
            ],
            bufs[b],
            sems[b],
        )

    GROUPS_PER_IT = G * L
    ones = jnp.ones((L,), jnp.int32)

    def process(b):
        def proc(i, cc, _b=b):
            row = i // (CC // GROUPS_PER_IT)
            colbase = (i % (CC // GROUPS_PER_IT)) * GROUPS_PER_IT
            xs = [bufs[_b][row, pl.ds(colbase + j * L, L)] for j in range(G)]
            for x in xs:
                plsc.addupdate_scatter(hist, [x], ones)
            return cc

        lax.fori_loop(0, CHUNK // GROUPS_PER_IT, proc, 0)

    for b in range(3):
        chunk_copy(b, b).start()

    NB = NCH // 3

    def outer(g, carry):
        for b in range(3):
            c = 3 * g + b
            chunk_copy(c, b).wait()
            process(b)

            @pl.when(c + 3 < NCH)
            def _(_c=c, _b=b):
                chunk_copy(_c + 3, _b).start()

        return carry

    lax.fori_loop(0, NB, outer, 0)

    for b in range(NCH - 3 * NB):
        chunk_copy(3 * NB + b, b).wait()
        process(b)

    pltpu.sync_copy(hist, out_hbm.at[wid])


def _reduce_body(parts_ref, unary_ref, out_ref):
    s = jnp.sum(parts_ref[...], axis=0, keepdims=True).astype(jnp.float32)
    out_ref[0] = unary_ref[0] + s


_reduce = pl.pallas_call(
    _reduce_body,
    grid=(V_PAD // RCOLS,),
    in_specs=[
        pl.BlockSpec((NW, RCOLS), lambda i: (0, i)),
        pl.BlockSpec((1, 1, RCOLS), lambda i: (i, 0, 0)),
    ],
    out_specs=pl.BlockSpec((1, 1, RCOLS), lambda i: (i, 0, 0)),
    out_shape=jax.ShapeDtypeStruct((V_PAD // RCOLS, 1, RCOLS), jnp.float32),
)


@jax.jit
def kernel(text, unary_counts):
    parts = _hist_kernel(text)
    unary_pad = jnp.pad(unary_counts, (0, V_PAD - V)).reshape(
        V_PAD // RCOLS, 1, RCOLS
    )
    out = _reduce(parts, unary_pad)
    return out.reshape(-1)[:V]


# speedup vs baseline: 1.4622x; 1.0071x over previous
import functools

import jax
import jax.numpy as jnp
from jax import lax
from jax.experimental import pallas as pl
from jax.experimental.pallas import tpu as pltpu
from jax.experimental.pallas import tpu_sc as plsc

V = 100002
ROWS, COLS = 4096, 2048
N_TOKENS = ROWS * COLS
NC, NS, L = 2, 16, 16
NW = NC * NS
ROWS_W = ROWS // NW
CR, CC = 8, 1024
CHUNK = CR * CC
NCH = ROWS_W // CR * 2
G = 16
RCOLS = 25600
V_PAD = 102400

_mesh = plsc.VectorSubcoreMesh(core_axis_name="c", subcore_axis_name="s")


@functools.partial(
    pl.kernel,
    out_type=jax.ShapeDtypeStruct((NW, V_PAD // 128, 128), jnp.int32),
    mesh=_mesh,
    scratch_types=[
        pltpu.VMEM((CR, CC), jnp.int32),
        pltpu.VMEM((CR, CC), jnp.int32),
        pltpu.VMEM((CR, CC), jnp.int32),
        pltpu.VMEM((V_PAD // 128, 128), jnp.int32),
        pltpu.SemaphoreType.DMA,
        pltpu.SemaphoreType.DMA,
        pltpu.SemaphoreType.DMA,
    ],
    compiler_params=pltpu.CompilerParams(
        needs_layout_passes=False, use_tc_tiling_on_sc=True
    ),
)
def _hist_kernel(text_hbm, out_hbm, buf0, buf1, buf2, hist, sem0, sem1, sem2):
    wid = lax.axis_index("s") * NC + lax.axis_index("c")
    base_row = wid * ROWS_W
    bufs = (buf0, buf1, buf2)
    sems = (sem0, sem1, sem2)

    zeros = jnp.zeros((L,), jnp.int32)

    def zero_body(i, carry):
        hist[i // 8, pl.ds((i % 8) * L, L)] = zeros
        return carry

    lax.fori_loop(0, V_PAD // L, zero_body, 0, unroll=8)

    def chunk_copy(c, b):
        return pltpu.make_async_copy(
            text_hbm.at[
                pl.ds(base_row + (c // 2) * CR, CR), pl.ds((c % 2) * CC, CC)
            ],
            bufs[b],
            sems[b],
        )

    GROUPS_PER_IT = G * L
    ones = jnp.ones((L,), jnp.int32)

    def process(b):
        def proc(i, cc, _b=b):
            row = i // (CC // GROUPS_PER_IT)
            colbase = (i % (CC // GROUPS_PER_IT)) * GROUPS_PER_IT
            xs = [bufs[_b][row, pl.ds(colbase + j * L, L)] for j in range(G)]
            for x in xs:
                plsc.addupdate_scatter(
                    hist, [jnp.right_shift(x, 7), jnp.bitwise_and(x, 127)], ones
                )
            return cc

        lax.fori_loop(0, CHUNK // GROUPS_PER_IT, proc, 0)

    for b in range(3):
        chunk_copy(b, b).start()

    NB = NCH // 3

    def outer(g, carry):
        for b in range(3):
            c = 3 * g + b
            chunk_copy(c, b).wait()
            process(b)

            @pl.when(c + 3 < NCH)
            def _(_c=c, _b=b):
                chunk_copy(_c + 3, _b).start()

        return carry

    lax.fori_loop(0, NB, outer, 0)

    for b in range(NCH - 3 * NB):
        chunk_copy(3 * NB + b, b).wait()
        process(b)

    pltpu.sync_copy(hist, out_hbm.at[wid])


def _reduce_body(parts_ref, unary_ref, out_ref):
    s = jnp.sum(parts_ref[...], axis=0).astype(jnp.float32)
    out_ref[0] = unary_ref[0] + s


_reduce = pl.pallas_call(
    _reduce_body,
    grid=(V_PAD // RCOLS,),
    in_specs=[
        pl.BlockSpec((NW, RCOLS // 128, 128), lambda i: (0, i, 0)),
        pl.BlockSpec((1, RCOLS // 128, 128), lambda i: (i, 0, 0)),
    ],
    out_specs=pl.BlockSpec((1, RCOLS // 128, 128), lambda i: (i, 0, 0)),
    out_shape=jax.ShapeDtypeStruct(
        (V_PAD // RCOLS, RCOLS // 128, 128), jnp.float32
    ),
)


@jax.jit
def kernel(text, unary_counts):
    parts = _hist_kernel(text)
    unary_pad = jnp.pad(unary_counts, (0, V_PAD - V)).reshape(
        V_PAD // RCOLS, RCOLS // 128, 128
    )
    out = _reduce(parts, unary_pad)
    return out.reshape(-1)[:V]
